# Initial kernel scaffold; baseline (speedup 1.0000x reference)
import jax
import jax.numpy as jnp
from jax.experimental import pallas as pl


def kernel(x, edge_index, edge_attr, W0, We0, att_src0, att_dst0, att_edge0, bias0, gamma0, beta0, W1, We1, att_src1, att_dst1, att_edge1, bias1, gamma1, beta1):
    raise NotImplementedError("write your pallas kernel here")


# speedup vs baseline: 21.5919x; 21.5919x over previous
import functools

import jax
import jax.numpy as jnp
from jax import lax
from jax.experimental import pallas as pl
from jax.experimental.pallas import tpu as pltpu
from jax.experimental.pallas import tpu_sc as plsc

N = 10000
E = 320000
D = 128
DE = 16

NPAD = 10240
CH = 128
CHUNKS = 2560
EPAD = CHUNKS * CH
TILE_CHUNKS = 160
CORE_CHUNKS = 80
GROUPS = 20
ROWS_PER_TILE = NPAD // 16


_BLK = 1024
_GRID = NPAD // _BLK


def _pre_body(x_ref, w_ref, s_ref, d_ref, h_ref, as_ref, ad_ref):
  h = jnp.dot(x_ref[...], w_ref[...], preferred_element_type=jnp.float32)
  h_ref[...] = h
  as_ref[...] = (h * s_ref[...]).sum(-1)
  ad_ref[...] = (h * d_ref[...]).sum(-1)


def _tc_pre(xp, W, asrc, adst):
  return pl.pallas_call(
      _pre_body,
      grid=(_GRID,),
      in_specs=[
          pl.BlockSpec((_BLK, D), lambda i: (i, 0)),
          pl.BlockSpec((D, D), lambda i: (0, 0)),
          pl.BlockSpec((1, D), lambda i: (0, 0)),
          pl.BlockSpec((1, D), lambda i: (0, 0)),
      ],
      out_specs=[
          pl.BlockSpec((_BLK, D), lambda i: (i, 0)),
          pl.BlockSpec((_BLK,), lambda i: (i,)),
          pl.BlockSpec((_BLK,), lambda i: (i,)),
      ],
      out_shape=[
          jax.ShapeDtypeStruct((NPAD, D), jnp.float32),
          jax.ShapeDtypeStruct((NPAD,), jnp.float32),
          jax.ShapeDtypeStruct((NPAD,), jnp.float32),
      ],
  )(xp, W, asrc, adst)


def _norm(o, g, b):
  mu = o.mean(-1, keepdims=True)
  var = ((o - mu) ** 2).mean(-1, keepdims=True)
  return (o - mu) * lax.rsqrt(var + 1e-5) * g + b


def _mid_body(pa_ref, pb_ref, bias_ref, g_ref, be_ref, w_ref, s_ref, d_ref,
              h_ref, as_ref, ad_ref):
  o = pa_ref[...] + pb_ref[...] + bias_ref[...]
  xn = _norm(o, g_ref[...], be_ref[...])
  h = jnp.dot(xn, w_ref[...], preferred_element_type=jnp.float32)
  h_ref[...] = h
  as_ref[...] = (h * s_ref[...]).sum(-1)
  ad_ref[...] = (h * d_ref[...]).sum(-1)


def _tc_mid(pa, pb, bias, gamma, beta, W, asrc, adst):
  return pl.pallas_call(
      _mid_body,
      grid=(_GRID,),
      in_specs=[
          pl.BlockSpec((_BLK, D), lambda i: (i, 0)),
          pl.BlockSpec((_BLK, D), lambda i: (i, 0)),
          pl.BlockSpec((1, D), lambda i: (0, 0)),
          pl.BlockSpec((1, D), lambda i: (0, 0)),
          pl.BlockSpec((1, D), lambda i: (0, 0)),
          pl.BlockSpec((D, D), lambda i: (0, 0)),
          pl.BlockSpec((1, D), lambda i: (0, 0)),
          pl.BlockSpec((1, D), lambda i: (0, 0)),
      ],
      out_specs=[
          pl.BlockSpec((_BLK, D), lambda i: (i, 0)),
          pl.BlockSpec((_BLK,), lambda i: (i,)),
          pl.BlockSpec((_BLK,), lambda i: (i,)),
      ],
      out_shape=[
          jax.ShapeDtypeStruct((NPAD, D), jnp.float32),
          jax.ShapeDtypeStruct((NPAD,), jnp.float32),
          jax.ShapeDtypeStruct((NPAD,), jnp.float32),
      ],
  )(pa, pb, bias, gamma, beta, W, asrc, adst)


def _fin_body(pa_ref, pb_ref, bias_ref, g_ref, be_ref, y_ref):
  o = pa_ref[...] + pb_ref[...] + bias_ref[...]
  y_ref[...] = _norm(o, g_ref[...], be_ref[...])


def _tc_fin(pa, pb, bias, gamma, beta):
  return pl.pallas_call(
      _fin_body,
      grid=(_GRID,),
      in_specs=[
          pl.BlockSpec((_BLK, D), lambda i: (i, 0)),
          pl.BlockSpec((_BLK, D), lambda i: (i, 0)),
          pl.BlockSpec((1, D), lambda i: (0, 0)),
          pl.BlockSpec((1, D), lambda i: (0, 0)),
          pl.BlockSpec((1, D), lambda i: (0, 0)),
      ],
      out_specs=pl.BlockSpec((_BLK, D), lambda i: (i, 0)),
      out_shape=jax.ShapeDtypeStruct((NPAD, D), jnp.float32),
  )(pa, pb, bias, gamma, beta)


_EBLK = 4096
_EGRID = EPAD // _EBLK


def _ae_body(ea_ref, we0_ref, a0_ref, we1_ref, a1_ref, ae0_ref, ae1_ref):
  ea = ea_ref[...]
  ve0 = (we0_ref[...] * a0_ref[...]).sum(-1)
  ve1 = (we1_ref[...] * a1_ref[...]).sum(-1)
  ae0_ref[...] = (ea * ve0[None, :]).sum(-1)
  ae1_ref[...] = (ea * ve1[None, :]).sum(-1)


def _tc_ae(edge_attr, We0, ae0v, We1, ae1v):
  return pl.pallas_call(
      _ae_body,
      grid=(_EGRID,),
      in_specs=[
          pl.BlockSpec((_EBLK, DE), lambda i: (i, 0)),
          pl.BlockSpec((DE, D), lambda i: (0, 0)),
          pl.BlockSpec((1, D), lambda i: (0, 0)),
          pl.BlockSpec((DE, D), lambda i: (0, 0)),
          pl.BlockSpec((1, D), lambda i: (0, 0)),
      ],
      out_specs=[
          pl.BlockSpec((_EBLK,), lambda i: (i,)),
          pl.BlockSpec((_EBLK,), lambda i: (i,)),
      ],
      out_shape=[
          jax.ShapeDtypeStruct((EPAD,), jnp.float32),
          jax.ShapeDtypeStruct((EPAD,), jnp.float32),
      ],
  )(edge_attr, We0, ae0v, We1, ae1v)


def _softmax_body(as_hbm, ad_hbm, ae_hbm, src_hbm, dst_hbm, alpha_hbm,
                  as_l, ad_l, ez80, dst80, srcr, dstr, aer, ezr, aw,
                  dacc, dsem):
  cid = lax.axis_index("c")
  sid = lax.axis_index("s")
  base = sid * ROWS_PER_TILE
  zeros16 = jnp.zeros((16,), jnp.float32)

  @plsc.parallel_loop(0, ROWS_PER_TILE // 16, 1)
  def _(v):
    as_l[pl.ds(v * 16, 16)] = zeros16

  pltpu.sync_copy(as_l.at[pl.ds(0, ROWS_PER_TILE)],
                  dacc.at[pl.ds(base, ROWS_PER_TILE)])

  pltpu.sync_copy(as_hbm, as_l)
  pltpu.sync_copy(ad_hbm, ad_l)
  plsc.subcore_barrier()

  own_base = sid * TILE_CHUNKS + cid * CORE_CHUNKS

  def p1_group(t, _):
    @pl.when(t >= 1)
    def _():
      for _ in range(8):
        pltpu.make_async_copy(ezr.at[0], dacc.at[dstr.at[0]], dsem).wait()

    gbase = sid * TILE_CHUNKS + t * 8
    pltpu.sync_copy(src_hbm.at[pl.ds(gbase, 8)], srcr)
    pltpu.sync_copy(dst_hbm.at[pl.ds(gbase, 8)], dstr)
    pltpu.sync_copy(ae_hbm.at[pl.ds(gbase, 8)], aer)

    own = (t >= GROUPS // 2) == (cid == 1)
    for j in range(8):
      for i in range(8):
        sl = pl.ds(i * 16, 16)
        s16 = srcr[j, sl]
        d16 = dstr[j, sl]
        za = (plsc.load_gather(as_l, [s16]) + plsc.load_gather(ad_l, [d16])
              + aer[j, sl])
        z = jnp.where(za >= 0, za, za * jnp.float32(0.2))
        ezr[j, sl] = jnp.exp(z)

      row = t * 8 + j - cid * CORE_CHUNKS

      @pl.when(own)
      def _():
        for i in range(8):
          sl = pl.ds(i * 16, 16)
          ez80[row, sl] = ezr[j, sl]
          dst80[row, sl] = dstr[j, sl]

      pltpu.async_copy(ezr.at[j], dacc.at[dstr.at[j]], dsem, add=True)
    return 0

  lax.fori_loop(0, GROUPS, p1_group, 0)
  for _ in range(8):
    pltpu.make_async_copy(ezr.at[0], dacc.at[dstr.at[0]], dsem).wait()

  plsc.subcore_barrier()

  pltpu.sync_copy(dacc, ad_l)

  @plsc.parallel_loop(0, NPAD // 16, 1, unroll=4)
  def _(v):
    sl = pl.ds(v * 16, 16)
    dv = ad_l[sl]
    ad_l[sl] = jnp.float32(1.0) / (dv + jnp.float32(1e-16))

  def a_group(t2, _):
    for jj in range(8):
      row = t2 * 8 + jj
      for i in range(8):
        sl = pl.ds(i * 16, 16)
        d16 = dst80[row, sl]
        aw[jj, sl] = ez80[row, sl] * plsc.load_gather(ad_l, [d16])
    pltpu.sync_copy(aw, alpha_hbm.at[pl.ds(own_base + t2 * 8, 8)])
    return 0

  lax.fori_loop(0, CORE_CHUNKS // 8, a_group, 0)


def _sc_softmax(a_s, a_d, ae2d, src2d, dst2d):
  mesh = plsc.VectorSubcoreMesh(core_axis_name="c", subcore_axis_name="s")
  f = pl.kernel(
      _softmax_body,
      out_type=jax.ShapeDtypeStruct((CHUNKS, CH), jnp.float32),
      mesh=mesh,
      compiler_params=pltpu.CompilerParams(needs_layout_passes=False),
      scratch_types=[
          pltpu.VMEM((NPAD,), jnp.float32),
          pltpu.VMEM((NPAD,), jnp.float32),
          pltpu.VMEM((CORE_CHUNKS, CH), jnp.float32),
          pltpu.VMEM((CORE_CHUNKS, CH), jnp.int32),
          pltpu.VMEM((8, CH), jnp.int32),
          pltpu.VMEM((8, CH), jnp.int32),
          pltpu.VMEM((8, CH), jnp.float32),
          pltpu.VMEM((8, CH), jnp.float32),
          pltpu.VMEM((8, CH), jnp.float32),
          pltpu.VMEM_SHARED((NPAD,), jnp.float32),
          pltpu.SemaphoreType.DMA,
      ],
  )
  return f(a_s, a_d, ae2d, src2d, dst2d)


def _agg_body(h_hbm, alpha_hbm, src_hbm, dst_hbm, out_hbm,
              rows, srcr, dstr, alphar,
              oacc, gsem0, gsem1, ssem0, ssem1, lsem0, lsem1):
  cid = lax.axis_index("c")
  sid = lax.axis_index("s")
  base = sid * ROWS_PER_TILE
  own_base = sid * TILE_CHUNKS + cid * CORE_CHUNKS
  zeros16 = jnp.zeros((16,), jnp.float32)

  @plsc.parallel_loop(0, CH, 1)
  def _(r):
    for k in range(D // 16):
      rows[0, r, pl.ds(k * 16, 16)] = zeros16

  for t in range(ROWS_PER_TILE // CH):
    pltpu.sync_copy(rows.at[0], oacc.at[pl.ds(base + t * CH, CH)])
  plsc.subcore_barrier()

  def load_trio(q, j, lsem):
    pltpu.async_copy(src_hbm.at[pl.ds(own_base + j, 1)], srcr.at[pl.ds(q, 1)],
                     lsem)
    pltpu.async_copy(dst_hbm.at[pl.ds(own_base + j, 1)], dstr.at[pl.ds(q, 1)],
                     lsem)
    pltpu.async_copy(alpha_hbm.at[pl.ds(own_base + j, 1)],
                     alphar.at[pl.ds(q, 1)], lsem)

  def wait_trio(lsem):
    pltpu.make_async_copy(src_hbm.at[pl.ds(own_base, 1)],
                          srcr.at[pl.ds(0, 1)], lsem).wait()
    pltpu.make_async_copy(dst_hbm.at[pl.ds(own_base, 1)],
                          dstr.at[pl.ds(0, 1)], lsem).wait()
    pltpu.make_async_copy(alpha_hbm.at[pl.ds(own_base, 1)],
                          alphar.at[pl.ds(0, 1)], lsem).wait()

  def wait_gather(slot, gsem):
    pltpu.make_async_copy(h_hbm.at[srcr.at[0]], rows.at[slot], gsem).wait()

  def wait_scatter(slot, ssem):
    pltpu.make_async_copy(rows.at[slot], oacc.at[dstr.at[0]], ssem).wait()

  def scale(slot, q):
    @plsc.parallel_loop(0, CH, 1, unroll=2)
    def _(r):
      bc = plsc.load_gather(alphar.at[q], [jnp.full((16,), r, jnp.int32)])
      for k in range(D // 16):
        sl = pl.ds(k * 16, 16)
        rows[slot, r, sl] = rows[slot, r, sl] * bc

  pltpu.sync_copy(src_hbm.at[pl.ds(own_base, 1)], srcr.at[pl.ds(0, 1)])
  pltpu.sync_copy(dst_hbm.at[pl.ds(own_base, 1)], dstr.at[pl.ds(0, 1)])
  pltpu.sync_copy(alpha_hbm.at[pl.ds(own_base, 1)], alphar.at[pl.ds(0, 1)])
  load_trio(1, 1, lsem1)
  pltpu.async_copy(h_hbm.at[srcr.at[0]], rows.at[0], gsem0)

  def process(j, r):
    gsem, gsem_o = (gsem0, gsem1) if r == 0 else (gsem1, gsem0)
    ssem, ssem_o = (ssem0, ssem1) if r == 0 else (ssem1, ssem0)
    lsem, lsem_o = (lsem0, lsem1) if r == 0 else (lsem1, lsem0)
    q = j % 4
    qn = (j + 1) % 4
    qnn = (j + 2) % 4

    @pl.when(j >= 1)
    def _():
      wait_scatter(1 - r, ssem_o)

    @pl.when(j + 1 < CORE_CHUNKS)
    def _():
      wait_trio(lsem_o)
      pltpu.async_copy(h_hbm.at[srcr.at[qn]], rows.at[1 - r], gsem_o)

    @pl.when(j + 2 < CORE_CHUNKS)
    def _():
      load_trio(qnn, j + 2, lsem)

    wait_gather(r, gsem)
    scale(r, q)
    pltpu.async_copy(rows.at[r], oacc.at[dstr.at[q]], ssem, add=True)

  def p2_body(i, _):
    process(2 * i, 0)
    process(2 * i + 1, 1)
    return 0

  lax.fori_loop(0, CORE_CHUNKS // 2, p2_body, 0)
  wait_scatter(1, ssem1)
  plsc.subcore_barrier()

  pltpu.sync_copy(oacc.at[pl.ds(base, ROWS_PER_TILE)],
                  out_hbm.at[cid].at[pl.ds(base, ROWS_PER_TILE)])


def _sc_agg(h, alpha2d, src2d, dst2d):
  mesh = plsc.VectorSubcoreMesh(core_axis_name="c", subcore_axis_name="s")
  f = pl.kernel(
      _agg_body,
      out_type=jax.ShapeDtypeStruct((2, NPAD, D), jnp.float32),
      mesh=mesh,
      compiler_params=pltpu.CompilerParams(needs_layout_passes=False),
      scratch_types=[
          pltpu.VMEM((2, CH, D), jnp.float32),
          pltpu.VMEM((4, CH), jnp.int32),
          pltpu.VMEM((4, CH), jnp.int32),
          pltpu.VMEM((4, CH), jnp.float32),
          pltpu.VMEM_SHARED((NPAD, D), jnp.float32),
          pltpu.SemaphoreType.DMA,
          pltpu.SemaphoreType.DMA,
          pltpu.SemaphoreType.DMA,
          pltpu.SemaphoreType.DMA,
          pltpu.SemaphoreType.DMA,
          pltpu.SemaphoreType.DMA,
      ],
  )
  return f(h, alpha2d, src2d, dst2d)


def _sc_layer(h, a_s, a_d, ae2d, src2d, dst2d):
  alpha2d = _sc_softmax(a_s, a_d, ae2d, src2d, dst2d)
  return _sc_agg(h, alpha2d, src2d, dst2d)


def kernel(x, edge_index, edge_attr,
           W0, We0, att_src0, att_dst0, att_edge0, bias0, gamma0, beta0,
           W1, We1, att_src1, att_dst1, att_edge1, bias1, gamma1, beta1):
  xp = jnp.pad(x, ((0, NPAD - N), (0, 0)))
  pad_idx = (jnp.arange(EPAD - E, dtype=jnp.int32) % N)
  src2d = jnp.concatenate([edge_index[0], pad_idx]).reshape(CHUNKS, CH)
  dst2d = jnp.concatenate([edge_index[1], pad_idx]).reshape(CHUNKS, CH)

  ea_pad = jnp.pad(edge_attr, ((0, EPAD - E), (0, 0)))
  ae0, ae1 = _tc_ae(ea_pad, We0, att_edge0.reshape(1, D),
                    We1, att_edge1.reshape(1, D))
  valid = jnp.arange(EPAD, dtype=jnp.int32) < E
  ae0 = jnp.where(valid, ae0, -1e30).reshape(CHUNKS, CH)
  ae1 = jnp.where(valid, ae1, -1e30).reshape(CHUNKS, CH)

  h0, as0, ad0 = _tc_pre(xp, W0, att_src0.reshape(1, D), att_dst0.reshape(1, D))
  op0 = _sc_layer(h0, as0, ad0, ae0, src2d, dst2d)
  h1, as1, ad1 = _tc_mid(op0[0], op0[1], bias0.reshape(1, D),
                         gamma0.reshape(1, D), beta0.reshape(1, D),
                         W1, att_src1.reshape(1, D), att_dst1.reshape(1, D))
  op1 = _sc_layer(h1, as1, ad1, ae1, src2d, dst2d)
  y = _tc_fin(op1[0], op1[1], bias1.reshape(1, D),
              gamma1.reshape(1, D), beta1.reshape(1, D))
  return y[:N]


# speedup vs baseline: 29.2608x; 1.3552x over previous
import functools

import jax
import jax.numpy as jnp
from jax import lax
from jax.experimental import pallas as pl
from jax.experimental.pallas import tpu as pltpu
from jax.experimental.pallas import tpu_sc as plsc

N = 10000
E = 320000
D = 128
DE = 16

NPAD = 10240
CH = 128
CHUNKS = 2560
EPAD = CHUNKS * CH
TILE_CHUNKS = 160
CORE_CHUNKS = 80
GROUPS = 20
ROWS_PER_TILE = NPAD // 16


_BLK = 1024
_GRID = NPAD // _BLK


def _pre_body(x_ref, w_ref, s_ref, d_ref, h_ref, as_ref, ad_ref):
  h = jnp.dot(x_ref[...], w_ref[...], preferred_element_type=jnp.float32)
  h_ref[...] = h
  as_ref[...] = (h * s_ref[...]).sum(-1)
  ad_ref[...] = (h * d_ref[...]).sum(-1)


def _tc_pre(xp, W, asrc, adst):
  return pl.pallas_call(
      _pre_body,
      grid=(_GRID,),
      in_specs=[
          pl.BlockSpec((_BLK, D), lambda i: (i, 0)),
          pl.BlockSpec((D, D), lambda i: (0, 0)),
          pl.BlockSpec((1, D), lambda i: (0, 0)),
          pl.BlockSpec((1, D), lambda i: (0, 0)),
      ],
      out_specs=[
          pl.BlockSpec((_BLK, D), lambda i: (i, 0)),
          pl.BlockSpec((_BLK,), lambda i: (i,)),
          pl.BlockSpec((_BLK,), lambda i: (i,)),
      ],
      out_shape=[
          jax.ShapeDtypeStruct((NPAD, D), jnp.float32),
          jax.ShapeDtypeStruct((NPAD,), jnp.float32),
          jax.ShapeDtypeStruct((NPAD,), jnp.float32),
      ],
  )(xp, W, asrc, adst)


def _norm(o, g, b):
  mu = o.mean(-1, keepdims=True)
  var = ((o - mu) ** 2).mean(-1, keepdims=True)
  return (o - mu) * lax.rsqrt(var + 1e-5) * g + b


def _mid_body(pa_ref, pb_ref, bias_ref, g_ref, be_ref, w_ref, s_ref, d_ref,
              h_ref, as_ref, ad_ref):
  o = pa_ref[...] + pb_ref[...] + bias_ref[...]
  xn = _norm(o, g_ref[...], be_ref[...])
  h = jnp.dot(xn, w_ref[...], preferred_element_type=jnp.float32)
  h_ref[...] = h
  as_ref[...] = (h * s_ref[...]).sum(-1)
  ad_ref[...] = (h * d_ref[...]).sum(-1)


def _tc_mid(pa, pb, bias, gamma, beta, W, asrc, adst):
  return pl.pallas_call(
      _mid_body,
      grid=(_GRID,),
      in_specs=[
          pl.BlockSpec((_BLK, D), lambda i: (i, 0)),
          pl.BlockSpec((_BLK, D), lambda i: (i, 0)),
          pl.BlockSpec((1, D), lambda i: (0, 0)),
          pl.BlockSpec((1, D), lambda i: (0, 0)),
          pl.BlockSpec((1, D), lambda i: (0, 0)),
          pl.BlockSpec((D, D), lambda i: (0, 0)),
          pl.BlockSpec((1, D), lambda i: (0, 0)),
          pl.BlockSpec((1, D), lambda i: (0, 0)),
      ],
      out_specs=[
          pl.BlockSpec((_BLK, D), lambda i: (i, 0)),
          pl.BlockSpec((_BLK,), lambda i: (i,)),
          pl.BlockSpec((_BLK,), lambda i: (i,)),
      ],
      out_shape=[
          jax.ShapeDtypeStruct((NPAD, D), jnp.float32),
          jax.ShapeDtypeStruct((NPAD,), jnp.float32),
          jax.ShapeDtypeStruct((NPAD,), jnp.float32),
      ],
  )(pa, pb, bias, gamma, beta, W, asrc, adst)


def _fin_body(pa_ref, pb_ref, bias_ref, g_ref, be_ref, y_ref):
  o = pa_ref[...] + pb_ref[...] + bias_ref[...]
  y_ref[...] = _norm(o, g_ref[...], be_ref[...])


def _tc_fin(pa, pb, bias, gamma, beta):
  return pl.pallas_call(
      _fin_body,
      grid=(_GRID,),
      in_specs=[
          pl.BlockSpec((_BLK, D), lambda i: (i, 0)),
          pl.BlockSpec((_BLK, D), lambda i: (i, 0)),
          pl.BlockSpec((1, D), lambda i: (0, 0)),
          pl.BlockSpec((1, D), lambda i: (0, 0)),
          pl.BlockSpec((1, D), lambda i: (0, 0)),
      ],
      out_specs=pl.BlockSpec((_BLK, D), lambda i: (i, 0)),
      out_shape=jax.ShapeDtypeStruct((NPAD, D), jnp.float32),
  )(pa, pb, bias, gamma, beta)


_PACK = D // DE
_ERS = E // _PACK
_EBLK = 4000
_EGRID = _ERS // _EBLK


def _ae_body(ea_ref, we0_ref, a0_ref, we1_ref, a1_ref, ae0_ref, ae1_ref):
  r = lax.broadcasted_iota(jnp.int32, (D, _PACK), 0)
  g = lax.broadcasted_iota(jnp.int32, (D, _PACK), 1)
  mask = ((r // DE) == g).astype(jnp.float32)
  ea = ea_ref[...]
  for we_ref, a_ref, ae_ref in ((we0_ref, a0_ref, ae0_ref),
                                (we1_ref, a1_ref, ae1_ref)):
    wet = jnp.concatenate([we_ref[...]] * _PACK, axis=0)
    vt = (wet * a_ref[...]).sum(-1)
    ae_ref[...] = jnp.dot(ea, vt[:, None] * mask,
                          precision=lax.Precision.HIGHEST,
                          preferred_element_type=jnp.float32)


def _tc_ae(ea_rs, We0, ae0v, We1, ae1v):
  return pl.pallas_call(
      _ae_body,
      grid=(_EGRID,),
      in_specs=[
          pl.BlockSpec((_EBLK, D), lambda i: (i, 0)),
          pl.BlockSpec((DE, D), lambda i: (0, 0)),
          pl.BlockSpec((1, D), lambda i: (0, 0)),
          pl.BlockSpec((DE, D), lambda i: (0, 0)),
          pl.BlockSpec((1, D), lambda i: (0, 0)),
      ],
      out_specs=[
          pl.BlockSpec((_EBLK, _PACK), lambda i: (i, 0)),
          pl.BlockSpec((_EBLK, _PACK), lambda i: (i, 0)),
      ],
      out_shape=[
          jax.ShapeDtypeStruct((_ERS, _PACK), jnp.float32),
          jax.ShapeDtypeStruct((_ERS, _PACK), jnp.float32),
      ],
  )(ea_rs, We0, ae0v, We1, ae1v)


def _softmax_body(as_hbm, ad_hbm, ae_hbm, src_hbm, dst_hbm, alpha_hbm,
                  as_l, ad_l, ez80, dst80, srcr, dstr, aer, ezr, aw,
                  dacc, dsem):
  cid = lax.axis_index("c")
  sid = lax.axis_index("s")
  base = sid * ROWS_PER_TILE
  zeros16 = jnp.zeros((16,), jnp.float32)

  @plsc.parallel_loop(0, ROWS_PER_TILE // 16, 1)
  def _(v):
    as_l[pl.ds(v * 16, 16)] = zeros16

  pltpu.sync_copy(as_l.at[pl.ds(0, ROWS_PER_TILE)],
                  dacc.at[pl.ds(base, ROWS_PER_TILE)])

  pltpu.sync_copy(as_hbm, as_l)
  pltpu.sync_copy(ad_hbm, ad_l)
  plsc.subcore_barrier()

  own_base = sid * TILE_CHUNKS + cid * CORE_CHUNKS

  def p1_group(t, _):
    @pl.when(t >= 1)
    def _():
      for _ in range(8):
        pltpu.make_async_copy(ezr.at[0], dacc.at[dstr.at[0]], dsem).wait()

    gbase = sid * TILE_CHUNKS + t * 8
    pltpu.sync_copy(src_hbm.at[pl.ds(gbase, 8)], srcr)
    pltpu.sync_copy(dst_hbm.at[pl.ds(gbase, 8)], dstr)
    pltpu.sync_copy(ae_hbm.at[pl.ds(gbase, 8)], aer)

    own = (t >= GROUPS // 2) == (cid == 1)
    for j in range(8):
      for i in range(8):
        sl = pl.ds(i * 16, 16)
        s16 = srcr[j, sl]
        d16 = dstr[j, sl]
        za = (plsc.load_gather(as_l, [s16]) + plsc.load_gather(ad_l, [d16])
              + aer[j, sl])
        z = jnp.where(za >= 0, za, za * jnp.float32(0.2))
        ezr[j, sl] = jnp.exp(z)

      row = t * 8 + j - cid * CORE_CHUNKS

      @pl.when(own)
      def _():
        for i in range(8):
          sl = pl.ds(i * 16, 16)
          ez80[row, sl] = ezr[j, sl]
          dst80[row, sl] = dstr[j, sl]

      pltpu.async_copy(ezr.at[j], dacc.at[dstr.at[j]], dsem, add=True)
    return 0

  lax.fori_loop(0, GROUPS, p1_group, 0)
  for _ in range(8):
    pltpu.make_async_copy(ezr.at[0], dacc.at[dstr.at[0]], dsem).wait()

  plsc.subcore_barrier()

  pltpu.sync_copy(dacc, ad_l)

  @plsc.parallel_loop(0, NPAD // 16, 1, unroll=4)
  def _(v):
    sl = pl.ds(v * 16, 16)
    dv = ad_l[sl]
    ad_l[sl] = jnp.float32(1.0) / (dv + jnp.float32(1e-16))

  def a_group(t2, _):
    for jj in range(8):
      row = t2 * 8 + jj
      for i in range(8):
        sl = pl.ds(i * 16, 16)
        d16 = dst80[row, sl]
        aw[jj, sl] = ez80[row, sl] * plsc.load_gather(ad_l, [d16])
    pltpu.sync_copy(aw, alpha_hbm.at[pl.ds(own_base + t2 * 8, 8)])
    return 0

  lax.fori_loop(0, CORE_CHUNKS // 8, a_group, 0)


def _sc_softmax(a_s, a_d, ae2d, src2d, dst2d):
  mesh = plsc.VectorSubcoreMesh(core_axis_name="c", subcore_axis_name="s")
  f = pl.kernel(
      _softmax_body,
      out_type=jax.ShapeDtypeStruct((CHUNKS, CH), jnp.float32),
      mesh=mesh,
      compiler_params=pltpu.CompilerParams(needs_layout_passes=False),
      scratch_types=[
          pltpu.VMEM((NPAD,), jnp.float32),
          pltpu.VMEM((NPAD,), jnp.float32),
          pltpu.VMEM((CORE_CHUNKS, CH), jnp.float32),
          pltpu.VMEM((CORE_CHUNKS, CH), jnp.int32),
          pltpu.VMEM((8, CH), jnp.int32),
          pltpu.VMEM((8, CH), jnp.int32),
          pltpu.VMEM((8, CH), jnp.float32),
          pltpu.VMEM((8, CH), jnp.float32),
          pltpu.VMEM((8, CH), jnp.float32),
          pltpu.VMEM_SHARED((NPAD,), jnp.float32),
          pltpu.SemaphoreType.DMA,
      ],
  )
  return f(a_s, a_d, ae2d, src2d, dst2d)


def _agg_body(h_hbm, alpha_hbm, src_hbm, dst_hbm, out_hbm,
              rows, srcr, dstr, alphar,
              oacc, gsem0, gsem1, ssem0, ssem1, lsem0, lsem1):
  cid = lax.axis_index("c")
  sid = lax.axis_index("s")
  base = sid * ROWS_PER_TILE
  own_base = sid * TILE_CHUNKS + cid * CORE_CHUNKS
  zeros16 = jnp.zeros((16,), jnp.float32)

  @plsc.parallel_loop(0, CH, 1)
  def _(r):
    for k in range(D // 16):
      rows[0, r, pl.ds(k * 16, 16)] = zeros16

  for t in range(ROWS_PER_TILE // CH):
    pltpu.sync_copy(rows.at[0], oacc.at[pl.ds(base + t * CH, CH)])
  plsc.subcore_barrier()

  def load_trio(q, j, lsem):
    pltpu.async_copy(src_hbm.at[pl.ds(own_base + j, 1)], srcr.at[pl.ds(q, 1)],
                     lsem)
    pltpu.async_copy(dst_hbm.at[pl.ds(own_base + j, 1)], dstr.at[pl.ds(q, 1)],
                     lsem)
    pltpu.async_copy(alpha_hbm.at[pl.ds(own_base + j, 1)],
                     alphar.at[pl.ds(q, 1)], lsem)

  def wait_trio(lsem):
    pltpu.make_async_copy(src_hbm.at[pl.ds(own_base, 1)],
                          srcr.at[pl.ds(0, 1)], lsem).wait()
    pltpu.make_async_copy(dst_hbm.at[pl.ds(own_base, 1)],
                          dstr.at[pl.ds(0, 1)], lsem).wait()
    pltpu.make_async_copy(alpha_hbm.at[pl.ds(own_base, 1)],
                          alphar.at[pl.ds(0, 1)], lsem).wait()

  def wait_gather(slot, gsem):
    pltpu.make_async_copy(h_hbm.at[srcr.at[0]], rows.at[slot], gsem).wait()

  def wait_scatter(slot, ssem):
    pltpu.make_async_copy(rows.at[slot], oacc.at[dstr.at[0]], ssem).wait()

  def scale(slot, q):
    @plsc.parallel_loop(0, CH, 1, unroll=2)
    def _(r):
      bc = plsc.load_gather(alphar.at[q], [jnp.full((16,), r, jnp.int32)])
      for k in range(D // 16):
        sl = pl.ds(k * 16, 16)
        rows[slot, r, sl] = rows[slot, r, sl] * bc

  pltpu.sync_copy(src_hbm.at[pl.ds(own_base, 1)], srcr.at[pl.ds(0, 1)])
  pltpu.sync_copy(dst_hbm.at[pl.ds(own_base, 1)], dstr.at[pl.ds(0, 1)])
  pltpu.sync_copy(alpha_hbm.at[pl.ds(own_base, 1)], alphar.at[pl.ds(0, 1)])
  load_trio(1, 1, lsem1)
  pltpu.async_copy(h_hbm.at[srcr.at[0]], rows.at[0], gsem0)

  def process(j, r):
    gsem, gsem_o = (gsem0, gsem1) if r == 0 else (gsem1, gsem0)
    ssem, ssem_o = (ssem0, ssem1) if r == 0 else (ssem1, ssem0)
    lsem, lsem_o = (lsem0, lsem1) if r == 0 else (lsem1, lsem0)
    q = j % 4
    qn = (j + 1) % 4
    qnn = (j + 2) % 4

    @pl.when(j >= 1)
    def _():
      wait_scatter(1 - r, ssem_o)

    @pl.when(j + 1 < CORE_CHUNKS)
    def _():
      wait_trio(lsem_o)
      pltpu.async_copy(h_hbm.at[srcr.at[qn]], rows.at[1 - r], gsem_o)

    @pl.when(j + 2 < CORE_CHUNKS)
    def _():
      load_trio(qnn, j + 2, lsem)

    wait_gather(r, gsem)
    scale(r, q)
    pltpu.async_copy(rows.at[r], oacc.at[dstr.at[q]], ssem, add=True)

  def p2_body(i, _):
    process(2 * i, 0)
    process(2 * i + 1, 1)
    return 0

  lax.fori_loop(0, CORE_CHUNKS // 2, p2_body, 0)
  wait_scatter(1, ssem1)
  plsc.subcore_barrier()

  pltpu.sync_copy(oacc.at[pl.ds(base, ROWS_PER_TILE)],
                  out_hbm.at[cid].at[pl.ds(base, ROWS_PER_TILE)])


def _sc_agg(h, alpha2d, src2d, dst2d):
  mesh = plsc.VectorSubcoreMesh(core_axis_name="c", subcore_axis_name="s")
  f = pl.kernel(
      _agg_body,
      out_type=jax.ShapeDtypeStruct((2, NPAD, D), jnp.float32),
      mesh=mesh,
      compiler_params=pltpu.CompilerParams(needs_layout_passes=False),
      scratch_types=[
          pltpu.VMEM((2, CH, D), jnp.float32),
          pltpu.VMEM((4, CH), jnp.int32),
          pltpu.VMEM((4, CH), jnp.int32),
          pltpu.VMEM((4, CH), jnp.float32),
          pltpu.VMEM_SHARED((NPAD, D), jnp.float32),
          pltpu.SemaphoreType.DMA,
          pltpu.SemaphoreType.DMA,
          pltpu.SemaphoreType.DMA,
          pltpu.SemaphoreType.DMA,
          pltpu.SemaphoreType.DMA,
          pltpu.SemaphoreType.DMA,
      ],
  )
  return f(h, alpha2d, src2d, dst2d)


def _sc_layer(h, a_s, a_d, ae2d, src2d, dst2d):
  alpha2d = _sc_softmax(a_s, a_d, ae2d, src2d, dst2d)
  return _sc_agg(h, alpha2d, src2d, dst2d)


def kernel(x, edge_index, edge_attr,
           W0, We0, att_src0, att_dst0, att_edge0, bias0, gamma0, beta0,
           W1, We1, att_src1, att_dst1, att_edge1, bias1, gamma1, beta1):
  xp = jnp.pad(x, ((0, NPAD - N), (0, 0)))
  pad_idx = (jnp.arange(EPAD - E, dtype=jnp.int32) % N)
  src2d = jnp.concatenate([edge_index[0], pad_idx]).reshape(CHUNKS, CH)
  dst2d = jnp.concatenate([edge_index[1], pad_idx]).reshape(CHUNKS, CH)

  ea_rs = edge_attr.reshape(_ERS, D)
  ae0, ae1 = _tc_ae(ea_rs, We0, att_edge0.reshape(1, D),
                    We1, att_edge1.reshape(1, D))
  ae_pad = jnp.full((EPAD - E,), -1e30, jnp.float32)
  ae0 = jnp.concatenate([ae0.reshape(E), ae_pad]).reshape(CHUNKS, CH)
  ae1 = jnp.concatenate([ae1.reshape(E), ae_pad]).reshape(CHUNKS, CH)

  h0, as0, ad0 = _tc_pre(xp, W0, att_src0.reshape(1, D), att_dst0.reshape(1, D))
  op0 = _sc_layer(h0, as0, ad0, ae0, src2d, dst2d)
  h1, as1, ad1 = _tc_mid(op0[0], op0[1], bias0.reshape(1, D),
                         gamma0.reshape(1, D), beta0.reshape(1, D),
                         W1, att_src1.reshape(1, D), att_dst1.reshape(1, D))
  op1 = _sc_layer(h1, as1, ad1, ae1, src2d, dst2d)
  y = _tc_fin(op1[0], op1[1], bias1.reshape(1, D),
              gamma1.reshape(1, D), beta1.reshape(1, D))
  return y[:N]


# speedup vs baseline: 32.1029x; 1.0971x over previous
import functools

import jax
import jax.numpy as jnp
from jax import lax
from jax.experimental import pallas as pl
from jax.experimental.pallas import tpu as pltpu
from jax.experimental.pallas import tpu_sc as plsc

N = 10000
E = 320000
D = 128
DE = 16

NPAD = 10240
CH = 128
CHUNKS = 2560
EPAD = CHUNKS * CH
TILE_CHUNKS = 160
CORE_CHUNKS = 80
GROUPS = 20
ROWS_PER_TILE = NPAD // 16


_BLK = 1024
_GRID = NPAD // _BLK


def _pre_body(x_ref, w_ref, s_ref, d_ref, h_ref, as_ref, ad_ref):
  h = jnp.dot(x_ref[...], w_ref[...], preferred_element_type=jnp.float32)
  h_ref[...] = h
  as_ref[...] = (h * s_ref[...]).sum(-1)
  ad_ref[...] = (h * d_ref[...]).sum(-1)


def _tc_pre(xp, W, asrc, adst):
  return pl.pallas_call(
      _pre_body,
      grid=(_GRID,),
      in_specs=[
          pl.BlockSpec((_BLK, D), lambda i: (i, 0)),
          pl.BlockSpec((D, D), lambda i: (0, 0)),
          pl.BlockSpec((1, D), lambda i: (0, 0)),
          pl.BlockSpec((1, D), lambda i: (0, 0)),
      ],
      out_specs=[
          pl.BlockSpec((_BLK, D), lambda i: (i, 0)),
          pl.BlockSpec((_BLK,), lambda i: (i,)),
          pl.BlockSpec((_BLK,), lambda i: (i,)),
      ],
      out_shape=[
          jax.ShapeDtypeStruct((NPAD, D), jnp.float32),
          jax.ShapeDtypeStruct((NPAD,), jnp.float32),
          jax.ShapeDtypeStruct((NPAD,), jnp.float32),
      ],
  )(xp, W, asrc, adst)


def _norm(o, g, b):
  mu = o.mean(-1, keepdims=True)
  var = ((o - mu) ** 2).mean(-1, keepdims=True)
  return (o - mu) * lax.rsqrt(var + 1e-5) * g + b


def _mid_body(pa_ref, pb_ref, bias_ref, g_ref, be_ref, w_ref, s_ref, d_ref,
              h_ref, as_ref, ad_ref):
  o = pa_ref[...] + pb_ref[...] + bias_ref[...]
  xn = _norm(o, g_ref[...], be_ref[...])
  h = jnp.dot(xn, w_ref[...], preferred_element_type=jnp.float32)
  h_ref[...] = h
  as_ref[...] = (h * s_ref[...]).sum(-1)
  ad_ref[...] = (h * d_ref[...]).sum(-1)


def _tc_mid(pa, pb, bias, gamma, beta, W, asrc, adst):
  return pl.pallas_call(
      _mid_body,
      grid=(_GRID,),
      in_specs=[
          pl.BlockSpec((_BLK, D), lambda i: (i, 0)),
          pl.BlockSpec((_BLK, D), lambda i: (i, 0)),
          pl.BlockSpec((1, D), lambda i: (0, 0)),
          pl.BlockSpec((1, D), lambda i: (0, 0)),
          pl.BlockSpec((1, D), lambda i: (0, 0)),
          pl.BlockSpec((D, D), lambda i: (0, 0)),
          pl.BlockSpec((1, D), lambda i: (0, 0)),
          pl.BlockSpec((1, D), lambda i: (0, 0)),
      ],
      out_specs=[
          pl.BlockSpec((_BLK, D), lambda i: (i, 0)),
          pl.BlockSpec((_BLK,), lambda i: (i,)),
          pl.BlockSpec((_BLK,), lambda i: (i,)),
      ],
      out_shape=[
          jax.ShapeDtypeStruct((NPAD, D), jnp.float32),
          jax.ShapeDtypeStruct((NPAD,), jnp.float32),
          jax.ShapeDtypeStruct((NPAD,), jnp.float32),
      ],
  )(pa, pb, bias, gamma, beta, W, asrc, adst)


def _fin_body(pa_ref, pb_ref, bias_ref, g_ref, be_ref, y_ref):
  o = pa_ref[...] + pb_ref[...] + bias_ref[...]
  y_ref[...] = _norm(o, g_ref[...], be_ref[...])


def _tc_fin(pa, pb, bias, gamma, beta):
  return pl.pallas_call(
      _fin_body,
      grid=(_GRID,),
      in_specs=[
          pl.BlockSpec((_BLK, D), lambda i: (i, 0)),
          pl.BlockSpec((_BLK, D), lambda i: (i, 0)),
          pl.BlockSpec((1, D), lambda i: (0, 0)),
          pl.BlockSpec((1, D), lambda i: (0, 0)),
          pl.BlockSpec((1, D), lambda i: (0, 0)),
      ],
      out_specs=pl.BlockSpec((_BLK, D), lambda i: (i, 0)),
      out_shape=jax.ShapeDtypeStruct((NPAD, D), jnp.float32),
  )(pa, pb, bias, gamma, beta)


ECHUNKS = E // CH


def _ve_body(we0_ref, a0_ref, we1_ref, a1_ref, ve0_ref, ve1_ref):
  ve0_ref[...] = (we0_ref[...] * a0_ref[...]).sum(-1)
  ve1_ref[...] = (we1_ref[...] * a1_ref[...]).sum(-1)


def _tc_ve(We0, ae0v, We1, ae1v):
  return pl.pallas_call(
      _ve_body,
      out_shape=[
          jax.ShapeDtypeStruct((DE,), jnp.float32),
          jax.ShapeDtypeStruct((DE,), jnp.float32),
      ],
  )(We0, ae0v, We1, ae1v)


def _softmax_body(as_hbm, ad_hbm, ea_hbm, ve_hbm, src_hbm, dst_hbm, alpha_hbm,
                  as_l, ad_l, ez80, dst80, srcr, dstr, ezr, aw, eag, ver,
                  dacc, dsem, lsem0, lsem1):
  cid = lax.axis_index("c")
  sid = lax.axis_index("s")
  base = sid * ROWS_PER_TILE
  zeros16 = jnp.zeros((16,), jnp.float32)

  @plsc.parallel_loop(0, ROWS_PER_TILE // 16, 1)
  def _(v):
    as_l[pl.ds(v * 16, 16)] = zeros16

  pltpu.sync_copy(as_l.at[pl.ds(0, ROWS_PER_TILE)],
                  dacc.at[pl.ds(base, ROWS_PER_TILE)])

  pltpu.sync_copy(as_hbm, as_l)
  pltpu.sync_copy(ad_hbm, ad_l)
  pltpu.sync_copy(ve_hbm, ver)
  vebc = [plsc.load_gather(ver, [jnp.full((16,), f, jnp.int32)])
          for f in range(DE)]
  plsc.subcore_barrier()

  own_base = sid * TILE_CHUNKS + cid * CORE_CHUNKS

  def issue_loads(g, slot, lsem):
    gbase = sid * TILE_CHUNKS + g * 8
    pltpu.async_copy(src_hbm.at[pl.ds(gbase, 8)], srcr.at[slot], lsem)
    pltpu.async_copy(dst_hbm.at[pl.ds(gbase, 8)], dstr.at[slot], lsem)
    for f in range(DE):
      pltpu.async_copy(ea_hbm.at[f].at[pl.ds(gbase, 8)],
                       eag.at[slot, f], lsem)

  def wait_loads(lsem):
    for _ in range(DE + 2):
      pltpu.make_async_copy(src_hbm.at[pl.ds(0, 8)], srcr.at[0], lsem).wait()

  def drain8():
    for _ in range(8):
      pltpu.make_async_copy(ezr.at[0, 0], dacc.at[dstr.at[0, 0]], dsem).wait()

  def p1_compute(t, slot):
    gbase = sid * TILE_CHUNKS + t * 8
    own = (t >= GROUPS // 2) == (cid == 1)
    for j in range(8):
      valid = gbase + j < ECHUNKS
      for i in range(8):
        sl = pl.ds(i * 16, 16)
        s16 = srcr[slot, j, sl]
        d16 = dstr[slot, j, sl]
        ae = vebc[0] * eag[slot, 0, j, sl]
        for f in range(1, DE):
          ae = ae + vebc[f] * eag[slot, f, j, sl]
        za = (plsc.load_gather(as_l, [s16]) + plsc.load_gather(ad_l, [d16])
              + ae)
        za = jnp.where(valid, za, jnp.float32(-1e30))
        z = jnp.where(za >= 0, za, za * jnp.float32(0.2))
        ezr[slot, j, sl] = jnp.exp(z)

      row = t * 8 + j - cid * CORE_CHUNKS

      @pl.when(own)
      def _():
        for i in range(8):
          sl = pl.ds(i * 16, 16)
          ez80[row, sl] = ezr[slot, j, sl]
          dst80[row, sl] = dstr[slot, j, sl]

      pltpu.async_copy(ezr.at[slot, j], dacc.at[dstr.at[slot, j]], dsem,
                       add=True)

  issue_loads(0, 0, lsem0)

  def p1_body(u, _):
    g0 = 2 * u

    @pl.when(u > 0)
    def _():
      drain8()

    issue_loads(g0 + 1, 1, lsem1)
    wait_loads(lsem0)
    p1_compute(g0, 0)
    wait_loads(lsem1)
    p1_compute(g0 + 1, 1)

    @pl.when(u < GROUPS // 2 - 1)
    def _():
      drain8()
      issue_loads(g0 + 2, 0, lsem0)

    return 0

  lax.fori_loop(0, GROUPS // 2, p1_body, 0)
  for _ in range(16):
    pltpu.make_async_copy(ezr.at[0, 0], dacc.at[dstr.at[0, 0]], dsem).wait()

  plsc.subcore_barrier()

  pltpu.sync_copy(dacc, ad_l)

  @plsc.parallel_loop(0, NPAD // 16, 1, unroll=4)
  def _(v):
    sl = pl.ds(v * 16, 16)
    dv = ad_l[sl]
    ad_l[sl] = jnp.float32(1.0) / (dv + jnp.float32(1e-16))

  def a_group(t2, _):
    for jj in range(8):
      row = t2 * 8 + jj
      for i in range(8):
        sl = pl.ds(i * 16, 16)
        d16 = dst80[row, sl]
        aw[jj, sl] = ez80[row, sl] * plsc.load_gather(ad_l, [d16])
    pltpu.sync_copy(aw, alpha_hbm.at[pl.ds(own_base + t2 * 8, 8)])
    return 0

  lax.fori_loop(0, CORE_CHUNKS // 8, a_group, 0)


def _sc_softmax(a_s, a_d, ea_rs, ve, src2d, dst2d):
  mesh = plsc.VectorSubcoreMesh(core_axis_name="c", subcore_axis_name="s")
  f = pl.kernel(
      _softmax_body,
      out_type=jax.ShapeDtypeStruct((CHUNKS, CH), jnp.float32),
      mesh=mesh,
      compiler_params=pltpu.CompilerParams(needs_layout_passes=False),
      scratch_types=[
          pltpu.VMEM((NPAD,), jnp.float32),
          pltpu.VMEM((NPAD,), jnp.float32),
          pltpu.VMEM((CORE_CHUNKS, CH), jnp.float32),
          pltpu.VMEM((CORE_CHUNKS, CH), jnp.int32),
          pltpu.VMEM((2, 8, CH), jnp.int32),
          pltpu.VMEM((2, 8, CH), jnp.int32),
          pltpu.VMEM((2, 8, CH), jnp.float32),
          pltpu.VMEM((8, CH), jnp.float32),
          pltpu.VMEM((2, DE, 8, CH), jnp.float32),
          pltpu.VMEM((DE,), jnp.float32),
          pltpu.VMEM_SHARED((NPAD,), jnp.float32),
          pltpu.SemaphoreType.DMA,
          pltpu.SemaphoreType.DMA,
          pltpu.SemaphoreType.DMA,
      ],
  )
  return f(a_s, a_d, ea_rs, ve, src2d, dst2d)


def _agg_body(h_hbm, alpha_hbm, src_hbm, dst_hbm, out_hbm,
              rows, srcr, dstr, alphar,
              oacc, gsem0, gsem1, ssem0, ssem1, lsem0, lsem1):
  cid = lax.axis_index("c")
  sid = lax.axis_index("s")
  base = sid * ROWS_PER_TILE
  own_base = sid * TILE_CHUNKS + cid * CORE_CHUNKS
  zeros16 = jnp.zeros((16,), jnp.float32)

  @plsc.parallel_loop(0, CH, 1)
  def _(r):
    for k in range(D // 16):
      rows[0, r, pl.ds(k * 16, 16)] = zeros16

  for t in range(ROWS_PER_TILE // CH):
    pltpu.sync_copy(rows.at[0], oacc.at[pl.ds(base + t * CH, CH)])
  plsc.subcore_barrier()

  def load_trio(q, j, lsem):
    pltpu.async_copy(src_hbm.at[pl.ds(own_base + j, 1)], srcr.at[pl.ds(q, 1)],
                     lsem)
    pltpu.async_copy(dst_hbm.at[pl.ds(own_base + j, 1)], dstr.at[pl.ds(q, 1)],
                     lsem)
    pltpu.async_copy(alpha_hbm.at[pl.ds(own_base + j, 1)],
                     alphar.at[pl.ds(q, 1)], lsem)

  def wait_trio(lsem):
    pltpu.make_async_copy(src_hbm.at[pl.ds(own_base, 1)],
                          srcr.at[pl.ds(0, 1)], lsem).wait()
    pltpu.make_async_copy(dst_hbm.at[pl.ds(own_base, 1)],
                          dstr.at[pl.ds(0, 1)], lsem).wait()
    pltpu.make_async_copy(alpha_hbm.at[pl.ds(own_base, 1)],
                          alphar.at[pl.ds(0, 1)], lsem).wait()

  def wait_gather(slot, gsem):
    pltpu.make_async_copy(h_hbm.at[srcr.at[0]], rows.at[slot], gsem).wait()

  def wait_scatter(slot, ssem):
    pltpu.make_async_copy(rows.at[slot], oacc.at[dstr.at[0]], ssem).wait()

  def scale(slot, q):
    @plsc.parallel_loop(0, CH, 1, unroll=2)
    def _(r):
      bc = plsc.load_gather(alphar.at[q], [jnp.full((16,), r, jnp.int32)])
      for k in range(D // 16):
        sl = pl.ds(k * 16, 16)
        rows[slot, r, sl] = rows[slot, r, sl] * bc

  pltpu.sync_copy(src_hbm.at[pl.ds(own_base, 1)], srcr.at[pl.ds(0, 1)])
  pltpu.sync_copy(dst_hbm.at[pl.ds(own_base, 1)], dstr.at[pl.ds(0, 1)])
  pltpu.sync_copy(alpha_hbm.at[pl.ds(own_base, 1)], alphar.at[pl.ds(0, 1)])
  load_trio(1, 1, lsem1)
  pltpu.async_copy(h_hbm.at[srcr.at[0]], rows.at[0], gsem0)

  def process(j, r):
    gsem, gsem_o = (gsem0, gsem1) if r == 0 else (gsem1, gsem0)
    ssem, ssem_o = (ssem0, ssem1) if r == 0 else (ssem1, ssem0)
    lsem, lsem_o = (lsem0, lsem1) if r == 0 else (lsem1, lsem0)
    q = j % 4
    qn = (j + 1) % 4
    qnn = (j + 2) % 4

    @pl.when(j >= 1)
    def _():
      wait_scatter(1 - r, ssem_o)

    @pl.when(j + 1 < CORE_CHUNKS)
    def _():
      wait_trio(lsem_o)
      pltpu.async_copy(h_hbm.at[srcr.at[qn]], rows.at[1 - r], gsem_o)

    @pl.when(j + 2 < CORE_CHUNKS)
    def _():
      load_trio(qnn, j + 2, lsem)

    wait_gather(r, gsem)
    scale(r, q)
    pltpu.async_copy(rows.at[r], oacc.at[dstr.at[q]], ssem, add=True)

  def p2_body(i, _):
    process(2 * i, 0)
    process(2 * i + 1, 1)
    return 0

  lax.fori_loop(0, CORE_CHUNKS // 2, p2_body, 0)
  wait_scatter(1, ssem1)
  plsc.subcore_barrier()

  pltpu.sync_copy(oacc.at[pl.ds(base, ROWS_PER_TILE)],
                  out_hbm.at[cid].at[pl.ds(base, ROWS_PER_TILE)])


def _sc_agg(h, alpha2d, src2d, dst2d):
  mesh = plsc.VectorSubcoreMesh(core_axis_name="c", subcore_axis_name="s")
  f = pl.kernel(
      _agg_body,
      out_type=jax.ShapeDtypeStruct((2, NPAD, D), jnp.float32),
      mesh=mesh,
      compiler_params=pltpu.CompilerParams(needs_layout_passes=False),
      scratch_types=[
          pltpu.VMEM((2, CH, D), jnp.float32),
          pltpu.VMEM((4, CH), jnp.int32),
          pltpu.VMEM((4, CH), jnp.int32),
          pltpu.VMEM((4, CH), jnp.float32),
          pltpu.VMEM_SHARED((NPAD, D), jnp.float32),
          pltpu.SemaphoreType.DMA,
          pltpu.SemaphoreType.DMA,
          pltpu.SemaphoreType.DMA,
          pltpu.SemaphoreType.DMA,
          pltpu.SemaphoreType.DMA,
          pltpu.SemaphoreType.DMA,
      ],
  )
  return f(h, alpha2d, src2d, dst2d)


def _sc_layer(h, a_s, a_d, ea_rs, ve, src2d, dst2d):
  alpha2d = _sc_softmax(a_s, a_d, ea_rs, ve, src2d, dst2d)
  return _sc_agg(h, alpha2d, src2d, dst2d)


def kernel(x, edge_index, edge_attr,
           W0, We0, att_src0, att_dst0, att_edge0, bias0, gamma0, beta0,
           W1, We1, att_src1, att_dst1, att_edge1, bias1, gamma1, beta1):
  xp = jnp.pad(x, ((0, NPAD - N), (0, 0)))
  pad_idx = (jnp.arange(EPAD - E, dtype=jnp.int32) % N)
  src2d = jnp.concatenate([edge_index[0], pad_idx]).reshape(CHUNKS, CH)
  dst2d = jnp.concatenate([edge_index[1], pad_idx]).reshape(CHUNKS, CH)

  ea_rs = jnp.pad(edge_attr, ((0, EPAD - E), (0, 0))).T.reshape(DE, CHUNKS, CH)
  ve0, ve1 = _tc_ve(We0, att_edge0.reshape(1, D), We1, att_edge1.reshape(1, D))

  h0, as0, ad0 = _tc_pre(xp, W0, att_src0.reshape(1, D), att_dst0.reshape(1, D))
  op0 = _sc_layer(h0, as0, ad0, ea_rs, ve0, src2d, dst2d)
  h1, as1, ad1 = _tc_mid(op0[0], op0[1], bias0.reshape(1, D),
                         gamma0.reshape(1, D), beta0.reshape(1, D),
                         W1, att_src1.reshape(1, D), att_dst1.reshape(1, D))
  op1 = _sc_layer(h1, as1, ad1, ea_rs, ve1, src2d, dst2d)
  y = _tc_fin(op1[0], op1[1], bias1.reshape(1, D),
              gamma1.reshape(1, D), beta1.reshape(1, D))
  return y[:N]
